# baseline (speedup 1.0000x reference)
import functools

import jax
import jax.numpy as jnp
from jax.experimental import pallas as pl
from jax.experimental.pallas import tpu as pltpu

_QB = 512


def _chamfer_body(
    src_ref, flow_ref, tgt_ref, out_ref, rT_ref, colmin_ref, colex_ref,
    *, nq, inv_n, inv_m
):
    b = pl.program_id(0)
    i = pl.program_id(1)

    @pl.when(i == 0)
    def _prep_target():
        rm = tgt_ref[0]
        rT3 = jnp.transpose(rm, (1, 0))
        rT_ref[0:3, :] = rT3
        rx = rT3[0:1, :]
        ry = rT3[1:2, :]
        rz = rT3[2:3, :]
        rT_ref[3:4, :] = rx * rx + ry * ry + rz * rz

    q3 = src_ref[0] + flow_ref[0]
    qs = -2.0 * q3
    qx = q3[:, 0:1]
    qy = q3[:, 1:2]
    qz = q3[:, 2:3]
    qn = qx * qx + qy * qy + qz * qz

    rT = rT_ref[0:3, :]
    rn = rT_ref[3:4, :]

    qr2 = jax.lax.dot_general(
        qs,
        rT,
        (((1,), (0,)), ((), ())),
        preferred_element_type=jnp.float32,
    )
    t = qn + rn
    d2a = t + qr2

    dot2 = (
        qs[:, 0:1] * rT[0:1, :]
        + qs[:, 1:2] * rT[1:2, :]
        + qs[:, 2:3] * rT[2:3, :]
    )
    delta = dot2 - qr2

    inf = jnp.float32(jnp.inf)

    rowmin = jnp.min(d2a, axis=1, keepdims=True)
    drow = jnp.min(
        jnp.where(d2a == rowmin, delta, inf), axis=1, keepdims=True
    )
    ex1 = jnp.maximum(rowmin + drow, 0.0)
    part = jnp.sum(jnp.sqrt(ex1), keepdims=True)

    colmin_t = jnp.min(d2a, axis=0, keepdims=True)
    dcol = jnp.min(
        jnp.where(d2a == colmin_t, delta, inf), axis=0, keepdims=True
    )
    colex_t = jnp.maximum(colmin_t + dcol, 0.0)

    @pl.when(jnp.logical_and(b == 0, i == 0))
    def _init():
        out_ref[...] = jnp.zeros((1, 1), jnp.float32)

    @pl.when(i == 0)
    def _first():
        colmin_ref[...] = colmin_t
        colex_ref[...] = colex_t

    @pl.when(i != 0)
    def _rest():
        repl = colmin_t < colmin_ref[...]
        colex_ref[...] = jnp.where(repl, colex_t, colex_ref[...])
        colmin_ref[...] = jnp.where(repl, colmin_t, colmin_ref[...])

    out_ref[...] += part * inv_n

    @pl.when(i == nq - 1)
    def _last():
        out_ref[...] += jnp.sum(jnp.sqrt(colex_ref[...]), keepdims=True) * inv_m


@jax.jit
def kernel(pc_source, pc_target, pred_flow):
    B, N, _ = pc_source.shape
    M = pc_target.shape[1]

    nq = N // _QB
    out = pl.pallas_call(
        functools.partial(
            _chamfer_body, nq=nq, inv_n=1.0 / (B * N), inv_m=1.0 / (B * M)
        ),
        grid=(B, nq),
        in_specs=[
            pl.BlockSpec((1, _QB, 3), lambda b, i: (b, i, 0)),
            pl.BlockSpec((1, _QB, 3), lambda b, i: (b, i, 0)),
            pl.BlockSpec((1, M, 3), lambda b, i: (b, 0, 0)),
        ],
        out_specs=pl.BlockSpec((1, 1), lambda b, i: (0, 0)),
        out_shape=jax.ShapeDtypeStruct((1, 1), jnp.float32),
        scratch_shapes=[
            pltpu.VMEM((8, M), jnp.float32),
            pltpu.VMEM((1, M), jnp.float32),
            pltpu.VMEM((1, M), jnp.float32),
        ],
    )(pc_source, pred_flow, pc_target)
    return out.reshape(())


# speedup vs baseline: 1.0273x; 1.0273x over previous
import functools

import jax
import jax.numpy as jnp
from jax.experimental import pallas as pl
from jax.experimental.pallas import tpu as pltpu

_QB = 1024


def _chamfer_body(
    src_ref, flow_ref, tgt_ref, out_ref, rT_ref, colmin_ref, colex_ref,
    *, nq, inv_n, inv_m
):
    b = pl.program_id(0)
    i = pl.program_id(1)

    @pl.when(i == 0)
    def _prep_target():
        rm = tgt_ref[0]
        rT3 = jnp.transpose(rm, (1, 0))
        rT_ref[0:3, :] = rT3
        rx = rT3[0:1, :]
        ry = rT3[1:2, :]
        rz = rT3[2:3, :]
        rT_ref[3:4, :] = rx * rx + ry * ry + rz * rz

    q3 = src_ref[0] + flow_ref[0]
    qs = -2.0 * q3
    qx = q3[:, 0:1]
    qy = q3[:, 1:2]
    qz = q3[:, 2:3]
    qn = qx * qx + qy * qy + qz * qz

    rT = rT_ref[0:3, :]
    rn = rT_ref[3:4, :]

    qr2 = jax.lax.dot_general(
        qs,
        rT,
        (((1,), (0,)), ((), ())),
        preferred_element_type=jnp.float32,
    )
    t = qn + rn
    d2a = t + qr2

    dot2 = (
        qs[:, 0:1] * rT[0:1, :]
        + qs[:, 1:2] * rT[1:2, :]
        + qs[:, 2:3] * rT[2:3, :]
    )
    delta = dot2 - qr2

    inf = jnp.float32(jnp.inf)

    rowmin = jnp.min(d2a, axis=1, keepdims=True)
    drow = jnp.min(
        jnp.where(d2a == rowmin, delta, inf), axis=1, keepdims=True
    )
    ex1 = jnp.maximum(rowmin + drow, 0.0)
    part = jnp.sum(jnp.sqrt(ex1), keepdims=True)

    colmin_t = jnp.min(d2a, axis=0, keepdims=True)
    dcol = jnp.min(
        jnp.where(d2a == colmin_t, delta, inf), axis=0, keepdims=True
    )
    colex_t = jnp.maximum(colmin_t + dcol, 0.0)

    @pl.when(jnp.logical_and(b == 0, i == 0))
    def _init():
        out_ref[...] = jnp.zeros((1, 1), jnp.float32)

    @pl.when(i == 0)
    def _first():
        colmin_ref[...] = colmin_t
        colex_ref[...] = colex_t

    @pl.when(i != 0)
    def _rest():
        repl = colmin_t < colmin_ref[...]
        colex_ref[...] = jnp.where(repl, colex_t, colex_ref[...])
        colmin_ref[...] = jnp.where(repl, colmin_t, colmin_ref[...])

    out_ref[...] += part * inv_n

    @pl.when(i == nq - 1)
    def _last():
        out_ref[...] += jnp.sum(jnp.sqrt(colex_ref[...]), keepdims=True) * inv_m


@jax.jit
def kernel(pc_source, pc_target, pred_flow):
    B, N, _ = pc_source.shape
    M = pc_target.shape[1]

    nq = N // _QB
    out = pl.pallas_call(
        functools.partial(
            _chamfer_body, nq=nq, inv_n=1.0 / (B * N), inv_m=1.0 / (B * M)
        ),
        grid=(B, nq),
        in_specs=[
            pl.BlockSpec((1, _QB, 3), lambda b, i: (b, i, 0)),
            pl.BlockSpec((1, _QB, 3), lambda b, i: (b, i, 0)),
            pl.BlockSpec((1, M, 3), lambda b, i: (b, 0, 0)),
        ],
        out_specs=pl.BlockSpec((1, 1), lambda b, i: (0, 0)),
        out_shape=jax.ShapeDtypeStruct((1, 1), jnp.float32),
        scratch_shapes=[
            pltpu.VMEM((8, M), jnp.float32),
            pltpu.VMEM((1, M), jnp.float32),
            pltpu.VMEM((1, M), jnp.float32),
        ],
    )(pc_source, pred_flow, pc_target)
    return out.reshape(())
